# baseline (speedup 1.0000x reference)
import functools

import jax
import jax.numpy as jnp
from jax import lax
from jax.experimental import pallas as pl
from jax.experimental.pallas import tpu as pltpu
from jax.experimental.pallas import tpu_sc as plsc

HIDDEN = 128
EPS = 1e-5
LANES = 16
CHUNK = 128
ROWB = 800


def _tc_norm_body(nblk, tab, g, b, out):
    i = pl.program_id(0)
    x = tab[...]
    mean = jnp.mean(x, axis=1, keepdims=True)
    xc = x - mean
    var = jnp.mean(xc * xc, axis=1, keepdims=True)
    y = xc * lax.rsqrt(var + EPS) * g[...] + b[...]
    out[...] = jnp.where(i == nblk - 1, 0.0, y)


def _sc_body(table, ids, mask, out,
             idx_v, m_v, b0, b1, b2, b3,
             gs0, gs1, gs2, gs3, ws0, ws1, ws2, ws3):
    info = plsc.get_sparse_core_info()
    nc = info.num_cores
    wid = lax.axis_index("s") * nc + lax.axis_index("c")
    n_tok = ids.shape[0] * ids.shape[1]
    n_per_w = n_tok // (nc * info.num_subcores)
    n_chunks = n_per_w // CHUNK
    zero_row = table.shape[0] - ROWB
    base = pl.multiple_of(wid * n_per_w, CHUNK)
    base_row = pl.multiple_of(wid * n_chunks, 8)

    pltpu.sync_copy(ids.at[pl.ds(base_row, n_chunks)], idx_v)
    pltpu.sync_copy(mask.at[pl.ds(base_row, n_chunks)], m_v)

    @plsc.parallel_loop(0, n_chunks)
    def _remap(r):
        for j in range(CHUNK // LANES):
            sl = pl.ds(LANES * j, LANES)
            idx_v[r, sl] = (idx_v[r, sl] - zero_row) * m_v[r, sl] + zero_row

    def gather(c, buf, sem):
        pltpu.async_copy(table.at[idx_v.at[c]], buf, sem)

    def gather_wait(buf, sem):
        pltpu.make_async_copy(table.at[idx_v.at[0]], buf, sem).wait()

    def writeback(c, buf, sem):
        tok0 = pl.multiple_of(base + c * CHUNK, CHUNK)
        pltpu.async_copy(buf, out.at[pl.ds(tok0, CHUNK)], sem)

    def wb_wait(buf, sem):
        pltpu.make_async_copy(buf, out.at[pl.ds(0, CHUNK)], sem).wait()

    bufs = (b0, b1, b2, b3)
    gsems = (gs0, gs1, gs2, gs3)
    wsems = (ws0, ws1, ws2, ws3)

    gather(0, bufs[0], gsems[0])
    gather(1, bufs[1], gsems[1])

    def quad_body(j, _):
        for t in range(4):
            c = 4 * j + t
            gather_wait(bufs[t], gsems[t])
            writeback(c, bufs[t], wsems[t])
            t2 = (t + 2) % 4
            c2 = c + 2

            def prefetch(t2=t2, c2=c2):
                pl.when(c2 >= 4)(
                    functools.partial(wb_wait, bufs[t2], wsems[t2]))
                gather(c2, bufs[t2], gsems[t2])

            pl.when(c2 < n_chunks)(prefetch)
        return 0

    lax.fori_loop(0, n_chunks // 4, quad_body, 0)
    for t in range(4):
        wb_wait(bufs[t], wsems[t])


@jax.jit
def _run(emb_table, ln_gamma, ln_beta, ids_2d, mask_2d):
    n = ids_2d.shape[0] * ids_2d.shape[1]
    vocab = emb_table.shape[0]
    nblk = vocab // ROWB + 1

    norm_table = pl.pallas_call(
        functools.partial(_tc_norm_body, nblk),
        grid=(nblk,),
        in_specs=[
            pl.BlockSpec((ROWB, HIDDEN),
                         lambda i: (jnp.minimum(i, nblk - 2), 0)),
            pl.BlockSpec((1, HIDDEN), lambda i: (0, 0)),
            pl.BlockSpec((1, HIDDEN), lambda i: (0, 0)),
        ],
        out_specs=pl.BlockSpec((ROWB, HIDDEN), lambda i: (i, 0)),
        out_shape=jax.ShapeDtypeStruct((nblk * ROWB, HIDDEN), jnp.float32),
    )(emb_table, ln_gamma.reshape(1, HIDDEN), ln_beta.reshape(1, HIDDEN))

    mesh = plsc.VectorSubcoreMesh(core_axis_name="c", subcore_axis_name="s")
    info = plsc.get_sparse_core_info()
    n_per_w = n // (info.num_cores * info.num_subcores)
    k = pl.kernel(
        _sc_body,
        out_type=jax.ShapeDtypeStruct((n, HIDDEN), jnp.float32),
        mesh=mesh,
        compiler_params=pltpu.CompilerParams(needs_layout_passes=False),
        scratch_types=[
            pltpu.VMEM((n_per_w // CHUNK, CHUNK), jnp.int32),
            pltpu.VMEM((n_per_w // CHUNK, CHUNK), jnp.int32),
            pltpu.VMEM((CHUNK, HIDDEN), jnp.float32),
            pltpu.VMEM((CHUNK, HIDDEN), jnp.float32),
            pltpu.VMEM((CHUNK, HIDDEN), jnp.float32),
            pltpu.VMEM((CHUNK, HIDDEN), jnp.float32),
        ] + [pltpu.SemaphoreType.DMA] * 8,
    )
    return k(norm_table, ids_2d, mask_2d)


def kernel(emb_table, ln_gamma, ln_beta, input_ids, attention_mask):
    b, l = input_ids.shape
    ids_2d = input_ids.reshape(b * l // CHUNK, CHUNK)
    mask_2d = attention_mask.reshape(b * l // CHUNK, CHUNK)
    out = _run(emb_table, ln_gamma, ln_beta, ids_2d, mask_2d)
    return out.reshape(b, l, HIDDEN)


# speedup vs baseline: 36.9731x; 36.9731x over previous
import functools

import jax
import jax.numpy as jnp
from jax import lax
from jax.experimental import pallas as pl
from jax.experimental.pallas import tpu as pltpu
from jax.experimental.pallas import tpu_sc as plsc

HIDDEN = 128
EPS = 1e-5
LANES = 16
CHUNK = 128
ROWB = 1000
NZBLK = 2


def _tc_norm_body(nblk, tab, g, b, out):
    i = pl.program_id(0)
    x = tab[...]
    mean = jnp.mean(x, axis=1, keepdims=True)
    xc = x - mean
    var = jnp.mean(xc * xc, axis=1, keepdims=True)
    y = xc * lax.rsqrt(var + EPS) * g[...] + b[...]
    out[...] = jnp.where(i >= nblk - NZBLK, 0.0, y)


def _sc_body(table, ids, mask, out,
             idx_v, m_v, b0, b1, b2, b3,
             gs0, gs1, gs2, gs3, ws0, ws1, ws2, ws3):
    info = plsc.get_sparse_core_info()
    nc = info.num_cores
    wid = lax.axis_index("s") * nc + lax.axis_index("c")
    n_tok = ids.shape[0] * ids.shape[1]
    n_per_w = n_tok // (nc * info.num_subcores)
    n_chunks = n_per_w // CHUNK
    zero_base = table.shape[0] - NZBLK * ROWB
    base = pl.multiple_of(wid * n_per_w, CHUNK)
    base_row = pl.multiple_of(wid * n_chunks, 8)

    pltpu.sync_copy(ids.at[pl.ds(base_row, n_chunks)], idx_v)
    pltpu.sync_copy(mask.at[pl.ds(base_row, n_chunks)], m_v)
    zlane = zero_base + lax.iota(jnp.int32, LANES)

    @plsc.parallel_loop(0, n_chunks)
    def _remap(r):
        roff = (r & 7) * 128
        for j in range(CHUNK // LANES):
            sl = pl.ds(LANES * j, LANES)
            z = zlane + (roff + LANES * j)
            idx_v[r, sl] = (idx_v[r, sl] - z) * m_v[r, sl] + z

    def gather(c, buf, sem):
        pltpu.async_copy(table.at[idx_v.at[c]], buf, sem)

    def gather_wait(buf, sem):
        pltpu.make_async_copy(table.at[idx_v.at[0]], buf, sem).wait()

    def writeback(c, buf, sem):
        tok0 = pl.multiple_of(base + c * CHUNK, CHUNK)
        pltpu.async_copy(buf, out.at[pl.ds(tok0, CHUNK)], sem)

    def wb_wait(buf, sem):
        pltpu.make_async_copy(buf, out.at[pl.ds(0, CHUNK)], sem).wait()

    bufs = (b0, b1, b2, b3)
    gsems = (gs0, gs1, gs2, gs3)
    wsems = (ws0, ws1, ws2, ws3)

    gather(0, bufs[0], gsems[0])
    gather(1, bufs[1], gsems[1])

    def quad_body(j, _):
        for t in range(4):
            c = 4 * j + t
            gather_wait(bufs[t], gsems[t])
            writeback(c, bufs[t], wsems[t])
            t2 = (t + 2) % 4
            c2 = c + 2

            def prefetch(t2=t2, c2=c2):
                pl.when(c2 >= 4)(
                    functools.partial(wb_wait, bufs[t2], wsems[t2]))
                gather(c2, bufs[t2], gsems[t2])

            pl.when(c2 < n_chunks)(prefetch)
        return 0

    lax.fori_loop(0, n_chunks // 4, quad_body, 0)
    for t in range(4):
        wb_wait(bufs[t], wsems[t])


@jax.jit
def _run(emb_table, ln_gamma, ln_beta, ids_2d, mask_2d):
    n = ids_2d.shape[0] * ids_2d.shape[1]
    vocab = emb_table.shape[0]
    nblk = vocab // ROWB + NZBLK

    norm_table = pl.pallas_call(
        functools.partial(_tc_norm_body, nblk),
        grid=(nblk,),
        in_specs=[
            pl.BlockSpec((ROWB, HIDDEN),
                         lambda i: (jnp.minimum(i, nblk - NZBLK - 1), 0)),
            pl.BlockSpec((1, HIDDEN), lambda i: (0, 0)),
            pl.BlockSpec((1, HIDDEN), lambda i: (0, 0)),
        ],
        out_specs=pl.BlockSpec((ROWB, HIDDEN), lambda i: (i, 0)),
        out_shape=jax.ShapeDtypeStruct((nblk * ROWB, HIDDEN), jnp.float32),
    )(emb_table, ln_gamma.reshape(1, HIDDEN), ln_beta.reshape(1, HIDDEN))

    mesh = plsc.VectorSubcoreMesh(core_axis_name="c", subcore_axis_name="s")
    info = plsc.get_sparse_core_info()
    n_per_w = n // (info.num_cores * info.num_subcores)
    k = pl.kernel(
        _sc_body,
        out_type=jax.ShapeDtypeStruct((n, HIDDEN), jnp.float32),
        mesh=mesh,
        compiler_params=pltpu.CompilerParams(needs_layout_passes=False),
        scratch_types=[
            pltpu.VMEM((n_per_w // CHUNK, CHUNK), jnp.int32),
            pltpu.VMEM((n_per_w // CHUNK, CHUNK), jnp.int32),
            pltpu.VMEM((CHUNK, HIDDEN), jnp.float32),
            pltpu.VMEM((CHUNK, HIDDEN), jnp.float32),
            pltpu.VMEM((CHUNK, HIDDEN), jnp.float32),
            pltpu.VMEM((CHUNK, HIDDEN), jnp.float32),
        ] + [pltpu.SemaphoreType.DMA] * 8,
    )
    return k(norm_table, ids_2d, mask_2d)


def kernel(emb_table, ln_gamma, ln_beta, input_ids, attention_mask):
    b, l = input_ids.shape
    ids_2d = input_ids.reshape(b * l // CHUNK, CHUNK)
    mask_2d = attention_mask.reshape(b * l // CHUNK, CHUNK)
    out = _run(emb_table, ln_gamma, ln_beta, ids_2d, mask_2d)
    return out.reshape(b, l, HIDDEN)


# speedup vs baseline: 37.1416x; 1.0046x over previous
import functools

import jax
import jax.numpy as jnp
from jax import lax
from jax.experimental import pallas as pl
from jax.experimental.pallas import tpu as pltpu
from jax.experimental.pallas import tpu_sc as plsc

HIDDEN = 128
EPS = 1e-5
LANES = 16
CHUNK = 128
ROWB = 1000
NZBLK = 2


def _tc_norm_body(nblk, tab, g, b, out):
    i = pl.program_id(0)
    x = tab[...]
    mean = jnp.mean(x, axis=1, keepdims=True)
    xc = x - mean
    var = jnp.mean(xc * xc, axis=1, keepdims=True)
    y = xc * lax.rsqrt(var + EPS) * g[...] + b[...]
    out[...] = jnp.where(i >= nblk - NZBLK, 0.0, y)


def _sc_body(table, ids, mask, out,
             idx_v, m_v, b0, b1, b2, b3,
             gs0, gs1, gs2, gs3, ws0, ws1, ws2, ws3):
    info = plsc.get_sparse_core_info()
    nc = info.num_cores
    wid = lax.axis_index("s") * nc + lax.axis_index("c")
    n_tok = ids.shape[0] * ids.shape[1]
    n_per_w = n_tok // (nc * info.num_subcores)
    n_chunks = n_per_w // CHUNK
    zero_base = table.shape[0] - NZBLK * ROWB
    base = pl.multiple_of(wid * n_per_w, CHUNK)
    base_row = pl.multiple_of(wid * n_chunks, 8)

    pltpu.sync_copy(ids.at[pl.ds(base_row, n_chunks)], idx_v)
    pltpu.sync_copy(mask.at[pl.ds(base_row, n_chunks)], m_v)
    zlane = zero_base + lax.iota(jnp.int32, LANES)

    @plsc.parallel_loop(0, n_chunks)
    def _remap(r):
        roff = (r & 7) * 128
        for j in range(CHUNK // LANES):
            sl = pl.ds(LANES * j, LANES)
            z = zlane + (roff + LANES * j)
            idx_v[r, sl] = (idx_v[r, sl] - z) * m_v[r, sl] + z

    def gather(c, buf, sem):
        pltpu.async_copy(table.at[idx_v.at[c]], buf, sem)

    def gather_wait(buf, sem):
        pltpu.make_async_copy(table.at[idx_v.at[0]], buf, sem).wait()

    def writeback(c, buf, sem):
        tok0 = pl.multiple_of(base + c * CHUNK, CHUNK)
        pltpu.async_copy(buf, out.at[pl.ds(tok0, CHUNK)], sem)

    def wb_wait(buf, sem):
        pltpu.make_async_copy(buf, out.at[pl.ds(0, CHUNK)], sem).wait()

    bufs = (b0, b1, b2, b3)
    gsems = (gs0, gs1, gs2, gs3)
    wsems = (ws0, ws1, ws2, ws3)

    gather(0, bufs[0], gsems[0])
    gather(1, bufs[1], gsems[1])
    gather(2, bufs[2], gsems[2])

    def quad_body(j, _):
        for t in range(4):
            c = 4 * j + t
            gather_wait(bufs[t], gsems[t])
            writeback(c, bufs[t], wsems[t])
            t2 = (t + 3) % 4
            c2 = c + 3

            def prefetch(t2=t2, c2=c2):
                pl.when(c2 >= 4)(
                    functools.partial(wb_wait, bufs[t2], wsems[t2]))
                gather(c2, bufs[t2], gsems[t2])

            pl.when(c2 < n_chunks)(prefetch)
        return 0

    lax.fori_loop(0, n_chunks // 4, quad_body, 0)
    for t in range(4):
        wb_wait(bufs[t], wsems[t])


@jax.jit
def _run(emb_table, ln_gamma, ln_beta, ids_2d, mask_2d):
    n = ids_2d.shape[0] * ids_2d.shape[1]
    vocab = emb_table.shape[0]
    nblk = vocab // ROWB + NZBLK

    norm_table = pl.pallas_call(
        functools.partial(_tc_norm_body, nblk),
        grid=(nblk,),
        in_specs=[
            pl.BlockSpec((ROWB, HIDDEN),
                         lambda i: (jnp.minimum(i, nblk - NZBLK - 1), 0)),
            pl.BlockSpec((1, HIDDEN), lambda i: (0, 0)),
            pl.BlockSpec((1, HIDDEN), lambda i: (0, 0)),
        ],
        out_specs=pl.BlockSpec((ROWB, HIDDEN), lambda i: (i, 0)),
        out_shape=jax.ShapeDtypeStruct((nblk * ROWB, HIDDEN), jnp.float32),
    )(emb_table, ln_gamma.reshape(1, HIDDEN), ln_beta.reshape(1, HIDDEN))

    mesh = plsc.VectorSubcoreMesh(core_axis_name="c", subcore_axis_name="s")
    info = plsc.get_sparse_core_info()
    n_per_w = n // (info.num_cores * info.num_subcores)
    k = pl.kernel(
        _sc_body,
        out_type=jax.ShapeDtypeStruct((n, HIDDEN), jnp.float32),
        mesh=mesh,
        compiler_params=pltpu.CompilerParams(needs_layout_passes=False),
        scratch_types=[
            pltpu.VMEM((n_per_w // CHUNK, CHUNK), jnp.int32),
            pltpu.VMEM((n_per_w // CHUNK, CHUNK), jnp.int32),
            pltpu.VMEM((CHUNK, HIDDEN), jnp.float32),
            pltpu.VMEM((CHUNK, HIDDEN), jnp.float32),
            pltpu.VMEM((CHUNK, HIDDEN), jnp.float32),
            pltpu.VMEM((CHUNK, HIDDEN), jnp.float32),
        ] + [pltpu.SemaphoreType.DMA] * 8,
    )
    return k(norm_table, ids_2d, mask_2d)


def kernel(emb_table, ln_gamma, ln_beta, input_ids, attention_mask):
    b, l = input_ids.shape
    ids_2d = input_ids.reshape(b * l // CHUNK, CHUNK)
    mask_2d = attention_mask.reshape(b * l // CHUNK, CHUNK)
    out = _run(emb_table, ln_gamma, ln_beta, ids_2d, mask_2d)
    return out.reshape(b, l, HIDDEN)
